# Initial kernel scaffold; baseline (speedup 1.0000x reference)
import jax
import jax.numpy as jnp
from jax.experimental import pallas as pl


def kernel(x, token_table, pos_table):
    raise NotImplementedError("write your pallas kernel here")


# speedup vs baseline: 3.2928x; 3.2928x over previous
import jax
import jax.numpy as jnp
from jax import lax
from jax.experimental import pallas as pl
from jax.experimental.pallas import tpu as pltpu
from jax.experimental.pallas import tpu_sc as plsc

HIDDEN = 64
MAX_LEN = 200
BATCH = 4096
LANES = 16
NUM_CORES = 2
NUM_SUBCORES = 16
NUM_WORKERS = NUM_CORES * NUM_SUBCORES
ROWS_PER_WORKER = BATCH // NUM_WORKERS
SPLIT_A = 104
SPLIT_B = MAX_LEN - SPLIT_A


def _body(x_hbm, tok_hbm, pos_hbm, out_hbm, idx_v, pos_v, buf, gsem):
    wid = lax.axis_index("s") * NUM_CORES + lax.axis_index("c")
    row0 = wid * ROWS_PER_WORKER

    pltpu.sync_copy(x_hbm.at[pl.ds(row0, ROWS_PER_WORKER)], idx_v)
    pltpu.sync_copy(pos_hbm, pos_v)

    def chunk(b, carry):
        c1 = pltpu.async_copy(
            tok_hbm.at[idx_v.at[b, pl.ds(0, SPLIT_A)]],
            buf.at[pl.ds(0, SPLIT_A)], gsem)
        c2 = pltpu.async_copy(
            tok_hbm.at[idx_v.at[b, pl.ds(SPLIT_A, SPLIT_B)]],
            buf.at[pl.ds(SPLIT_A, SPLIT_B)], gsem)
        c1.wait()
        c2.wait()

        def add_row(r, c2_):
            for j in range(HIDDEN // LANES):
                plsc.addupdate(buf.at[r, pl.ds(LANES * j, LANES)],
                               pos_v[r, pl.ds(LANES * j, LANES)])
            return c2_
        lax.fori_loop(0, MAX_LEN, add_row, 0)

        pltpu.sync_copy(buf, out_hbm.at[row0 + b])
        return carry

    lax.fori_loop(0, ROWS_PER_WORKER, chunk, 0)


def kernel(x, token_table, pos_table):
    mesh = plsc.VectorSubcoreMesh(
        core_axis_name="c", subcore_axis_name="s",
        num_cores=NUM_CORES, num_subcores=NUM_SUBCORES)
    f = pl.kernel(
        _body,
        out_type=jax.ShapeDtypeStruct((BATCH, MAX_LEN, HIDDEN), jnp.float32),
        mesh=mesh,
        compiler_params=pltpu.CompilerParams(use_tc_tiling_on_sc=False),
        scratch_types=[
            pltpu.VMEM((ROWS_PER_WORKER, MAX_LEN), jnp.int32),
            pltpu.VMEM((MAX_LEN, HIDDEN), jnp.float32),
            pltpu.VMEM((MAX_LEN, HIDDEN), jnp.float32),
            pltpu.SemaphoreType.DMA,
        ],
    )
    return f(x.astype(jnp.int32), token_table, pos_table)


# speedup vs baseline: 4.2387x; 1.2873x over previous
import jax
import jax.numpy as jnp
from jax import lax
from jax.experimental import pallas as pl
from jax.experimental.pallas import tpu as pltpu
from jax.experimental.pallas import tpu_sc as plsc

HIDDEN = 64
MAX_LEN = 200
BATCH = 4096
LANES = 16
NUM_CORES = 2
NUM_SUBCORES = 16
NUM_WORKERS = NUM_CORES * NUM_SUBCORES
ROWS_PER_WORKER = BATCH // NUM_WORKERS
SPLIT = (104, 96)
NBUF = 8
DEPTH = 6
CHUNKS = 2 * ROWS_PER_WORKER


def _body(x_hbm, tok_hbm, pos_hbm, out_hbm, idx_v, pos_v, *rest):
    bufs = rest[0:NBUF]
    gsem = rest[NBUF:2 * NBUF]
    wsem = rest[2 * NBUF:3 * NBUF]

    wid = lax.axis_index("s") * NUM_CORES + lax.axis_index("c")
    row0 = wid * ROWS_PER_WORKER

    pltpu.sync_copy(x_hbm.at[pl.ds(row0, ROWS_PER_WORKER)], idx_v)
    pltpu.sync_copy(pos_hbm, pos_v)

    def issue_gather(br, k):
        p = k % 2
        sz = SPLIT[p]
        pltpu.async_copy(
            tok_hbm.at[idx_v.at[br, pl.ds(104 * p, sz)]],
            bufs[k].at[pl.ds(0, sz)], gsem[k])

    for c in range(DEPTH):
        issue_gather(c // 2, c)

    @pl.loop(0, CHUNKS, step=NBUF)
    def _slots(o):
        o2 = o // 2
        for k in range(NBUF):
            b = o + k
            p = k % 2
            sz = SPLIT[p]
            br = o2 + (k // 2)
            pltpu.make_async_copy(
                tok_hbm.at[pl.ds(0, sz)], bufs[k].at[pl.ds(0, sz)],
                gsem[k]).wait()

            @pl.loop(0, sz // 2)
            def _add(r2, _k=k, _p=p):
                r = 2 * r2
                for dr in range(2):
                    for j in range(HIDDEN // LANES):
                        plsc.addupdate(
                            bufs[_k].at[r + dr, pl.ds(LANES * j, LANES)],
                            pos_v[104 * _p + r + dr, pl.ds(LANES * j, LANES)])

            pltpu.async_copy(
                bufs[k].at[pl.ds(0, sz)],
                out_hbm.at[row0 + br, pl.ds(104 * p, sz)], wsem[k])

            kj = (k + DEPTH) % NBUF
            pj = kj % 2
            szj = SPLIT[pj]

            @pl.when(jnp.logical_and(b >= NBUF - DEPTH, b < CHUNKS - DEPTH))
            def _w(_kj=kj, _pj=pj, _szj=szj):
                pltpu.make_async_copy(
                    bufs[_kj].at[pl.ds(0, _szj)],
                    out_hbm.at[0, pl.ds(104 * _pj, _szj)], wsem[_kj]).wait()

            @pl.when(b < CHUNKS - DEPTH)
            def _g(_kj=kj, _brj=o2 + (k + DEPTH) // 2):
                issue_gather(_brj, _kj)

    for k in range(NBUF):
        p = k % 2
        sz = SPLIT[p]
        pltpu.make_async_copy(
            bufs[k].at[pl.ds(0, sz)],
            out_hbm.at[0, pl.ds(104 * p, sz)], wsem[k]).wait()


def kernel(x, token_table, pos_table):
    mesh = plsc.VectorSubcoreMesh(
        core_axis_name="c", subcore_axis_name="s",
        num_cores=NUM_CORES, num_subcores=NUM_SUBCORES)
    scratch = (
        [pltpu.VMEM((ROWS_PER_WORKER, MAX_LEN), jnp.int32)]
        + [pltpu.VMEM((MAX_LEN, HIDDEN), jnp.float32)]
        + [pltpu.VMEM((SPLIT[0], HIDDEN), jnp.float32)] * NBUF
        + [pltpu.SemaphoreType.DMA] * (2 * NBUF)
    )
    f = pl.kernel(
        _body,
        out_type=jax.ShapeDtypeStruct((BATCH, MAX_LEN, HIDDEN), jnp.float32),
        mesh=mesh,
        compiler_params=pltpu.CompilerParams(use_tc_tiling_on_sc=False),
        scratch_types=scratch,
    )
    return f(x.astype(jnp.int32), token_table, pos_table)
